# baseline (speedup 1.0000x reference)
import functools
import math

import jax
import jax.numpy as jnp
from jax import lax
from jax.experimental import pallas as pl
from jax.experimental.pallas import tpu as pltpu
from jax.experimental.pallas import tpu_sc as plsc

N = 10000
E = 160000
DIM = 256
EDIM = 16
H = 8
C = DIM // H
MAXDEG = 512

BN = 2000
BE = 2000

_INTERP = False

NC = 2
NS = 16
NW = NC * NS
BSC = 128
EW = 39 * BSC
ETAIL = NW * EW
RPW = 624
RTAIL = N - RPW * NS


def _sc_mesh():
    return plsc.VectorSubcoreMesh(core_axis_name="c", subcore_axis_name="s")


_SC_PARAMS = pltpu.CompilerParams(needs_layout_passes=False)


def _striped(copy_fn, sid):
    copy_fn(sid * RPW, RPW)

    @pl.when(sid == NS - 1)
    def _():
        copy_fn(RPW * NS, RTAIL)


def _edge_chunks(chunk_fn, wid):
    base = wid * EW
    lax.fori_loop(0, EW // BSC,
                  lambda t, _: (chunk_fn(base + t * BSC), 0)[1], 0)

    @pl.when(wid < 2)
    def _():
        chunk_fn(ETAIL + wid * BSC)


def _erf(x):
    s = jnp.sign(x)
    ax = jnp.abs(x)
    t = 1.0 / (1.0 + 0.3275911 * ax)
    poly = t * (0.254829592 + t * (-0.284496736 + t * (1.421413741
           + t * (-1.453152027 + t * 1.061405429))))
    return s * (1.0 - poly * jnp.exp(-ax * ax))


def _gelu(x):
    return 0.5 * x * (1.0 + _erf(x * 0.7071067811865476))


def _ln(x, g, b, eps=1e-5):
    mu = jnp.mean(x, axis=-1, keepdims=True)
    var = jnp.mean((x - mu) ** 2, axis=-1, keepdims=True)
    return (x - mu) * jax.lax.rsqrt(var + eps) * g + b


def _p0(src, zeros128, ones128):
    @functools.partial(
        pl.kernel,
        out_type=jax.ShapeDtypeStruct((2 * N, 128), jnp.float32),
        mesh=_sc_mesh(),
        compiler_params=_SC_PARAMS,
        scratch_types=[pltpu.VMEM((BSC,), jnp.int32),
                       pltpu.VMEM((BSC, 128), jnp.float32),
                       pltpu.VMEM_SHARED((N, 128), jnp.float32)])
    def body(src_h, zeros_h, ones_h, out_h, idx_v, ones_v, cnt_sh):
        cid = lax.axis_index("c")
        sid = lax.axis_index("s")
        wid = cid * NS + sid
        _striped(lambda r0, nr: pltpu.sync_copy(
            zeros_h.at[pl.ds(r0, nr)], cnt_sh.at[pl.ds(r0, nr)]), sid)
        pltpu.sync_copy(ones_h, ones_v)
        plsc.subcore_barrier()

        def chunk(cbase):
            pltpu.sync_copy(src_h.at[pl.ds(cbase, BSC)], idx_v)
            pltpu.sync_copy(ones_v, cnt_sh.at[idx_v], add=True)

        _edge_chunks(chunk, wid)
        plsc.subcore_barrier()
        _striped(lambda r0, nr: pltpu.sync_copy(
            cnt_sh.at[pl.ds(r0, nr)],
            out_h.at[pl.ds(cid * N + r0, nr)]), sid)

    return body(src, zeros128, ones128)


B2 = 64
NCH2 = E // B2
CW2 = NCH2 // NW
C2TAIL = CW2 * NW


def _p2a(q, k, ecat, src, dst):
    inv_sqrt_c = 1.0 / math.sqrt(float(C))

    @functools.partial(
        pl.kernel,
        out_type=jax.ShapeDtypeStruct((E, 16), jnp.float32),
        mesh=_sc_mesh(),
        compiler_params=_SC_PARAMS,
        scratch_types=[pltpu.VMEM((2, B2), jnp.int32),
                       pltpu.VMEM((2, B2), jnp.int32),
                       pltpu.VMEM((2, B2, 256), jnp.float32),
                       pltpu.VMEM((2, B2, 256), jnp.float32),
                       pltpu.VMEM((2, B2, 128), jnp.float32),
                       pltpu.VMEM((2, B2, 128), jnp.float32),
                       pltpu.VMEM((B2, 16), jnp.float32)]
        + [pltpu.SemaphoreType.DMA] * 12)
    def body(q_h, k_h, e_h, src_h, dst_h, ae_h,
             idxs2, idxd2, kv2, qv2, e0v2, e1v2, aev, *sems):
        cid = lax.axis_index("c")
        sid = lax.axis_index("s")
        wid = cid * NS + sid
        nch = CW2 + jnp.where(wid < NCH2 - C2TAIL, 1, 0)
        ssem = [sems[:4], sems[4:8]]
        isem = [sems[8:10], sems[10:12]]

        def cbase_of(t):
            return jnp.where(t < CW2, (wid * CW2 + t) * B2,
                             (C2TAIL + wid) * B2)

        def idx_fetch(t, b):
            cb = cbase_of(t)
            si, sd = isem[b]
            pltpu.async_copy(src_h.at[pl.ds(cb, B2)], idxs2.at[b], si)
            pltpu.async_copy(dst_h.at[pl.ds(cb, B2)], idxd2.at[b], sd)

        def idx_wait(b):
            si, sd = isem[b]
            pltpu.make_async_copy(src_h.at[pl.ds(0, B2)], idxs2.at[b],
                                  si).wait()
            pltpu.make_async_copy(dst_h.at[pl.ds(0, B2)], idxd2.at[b],
                                  sd).wait()

        def data_issue(t, b):
            cb = cbase_of(t)
            sk, sq, se0, se1 = ssem[b]
            pltpu.async_copy(k_h.at[idxs2.at[b]], kv2.at[b], sk)
            pltpu.async_copy(q_h.at[idxd2.at[b]], qv2.at[b], sq)
            pltpu.async_copy(e_h.at[pl.ds(cb, B2)], e0v2.at[b], se0)
            pltpu.async_copy(e_h.at[pl.ds(E + cb, B2)], e1v2.at[b], se1)

        def data_wait(b):
            sk, sq, se0, se1 = ssem[b]
            pltpu.make_async_copy(k_h.at[idxs2.at[b]], kv2.at[b], sk).wait()
            pltpu.make_async_copy(q_h.at[idxd2.at[b]], qv2.at[b], sq).wait()
            pltpu.make_async_copy(e_h.at[pl.ds(0, B2)], e0v2.at[b],
                                  se0).wait()
            pltpu.make_async_copy(e_h.at[pl.ds(0, B2)], e1v2.at[b],
                                  se1).wait()

        def compute(t, b):
            def row(i, _):
                lane = lax.iota(jnp.int32, 16)
                alpha = jnp.zeros((16,), jnp.float32)
                for h in range(H):
                    ev2 = e0v2 if h < 4 else e1v2
                    eoff = (h % 4) * 32
                    k0 = (kv2[b, i, pl.ds(h * 32, 16)]
                          + ev2[b, i, pl.ds(eoff, 16)])
                    k1 = (kv2[b, i, pl.ds(h * 32 + 16, 16)]
                          + ev2[b, i, pl.ds(eoff + 16, 16)])
                    p = (k0 * qv2[b, i, pl.ds(h * 32, 16)]
                         + k1 * qv2[b, i, pl.ds(h * 32 + 16, 16)])
                    t2 = jnp.sum(p)
                    alpha = jnp.where(lane == h, t2, alpha)
                ae = jnp.exp(alpha * inv_sqrt_c)
                aev[i, :] = ae
                return 0

            lax.fori_loop(0, B2, row, 0)
            pltpu.sync_copy(aev, ae_h.at[pl.ds(cbase_of(t), B2)])

        idx_fetch(0, 0)
        idx_wait(0)
        data_issue(0, 0)

        @pl.when(nch > 1)
        def _():
            idx_fetch(1, 1)

        def outer(o, _):
            for bslot in range(2):
                t = o * 2 + bslot

                @pl.when(t < nch)
                def _():
                    @pl.when(t + 1 < nch)
                    def _():
                        idx_wait(1 - bslot)
                        data_issue(t + 1, 1 - bslot)

                    data_wait(bslot)
                    compute(t, bslot)

                    @pl.when(t + 2 < nch)
                    def _():
                        idx_fetch(t + 2, bslot)
            return 0

        lax.fori_loop(0, (CW2 + 2) // 2, outer, 0)

    return body(q, k, ecat, src, dst)


def _p2b(ae, dst, zeros16):
    @functools.partial(
        pl.kernel,
        out_type=jax.ShapeDtypeStruct((2 * N, 128), jnp.float32),
        mesh=_sc_mesh(),
        compiler_params=_SC_PARAMS,
        scratch_types=[pltpu.VMEM((2, BSC), jnp.int32),
                       pltpu.VMEM((2, BSC, 16), jnp.float32),
                       pltpu.VMEM((BSC, 128), jnp.float32),
                       pltpu.VMEM_SHARED((N, 128), jnp.float32)]
        + [pltpu.SemaphoreType.DMA] * 4)
    def body(ae_h, dst_h, zeros_h, den_h, idxd2, aev2, aepad_v, den_sh,
             *sems):
        cid = lax.axis_index("c")
        sid = lax.axis_index("s")
        wid = cid * NS + sid
        _striped(lambda r0, nr: pltpu.sync_copy(
            zeros_h.at[pl.ds(r0, nr)], den_sh.at[pl.ds(r0, nr)]), sid)
        plsc.subcore_barrier()
        nch = 39 + jnp.where(wid < 2, 1, 0)
        ssem = [sems[:2], sems[2:]]

        def cbase_of(t):
            return jnp.where(t < 39, wid * EW + t * BSC, ETAIL + wid * BSC)

        def fetch(t, b):
            cb = cbase_of(t)
            si, sa = ssem[b]
            pltpu.async_copy(dst_h.at[pl.ds(cb, BSC)], idxd2.at[b], si)
            pltpu.async_copy(ae_h.at[pl.ds(cb, BSC)], aev2.at[b], sa)

        def wait(b):
            si, sa = ssem[b]
            pltpu.make_async_copy(dst_h.at[pl.ds(0, BSC)], idxd2.at[b],
                                  si).wait()
            pltpu.make_async_copy(ae_h.at[pl.ds(0, BSC)], aev2.at[b],
                                  sa).wait()

        fetch(0, 0)

        def outer(o, _):
            for bslot in range(2):
                t = o * 2 + bslot

                @pl.when(t < nch)
                def _():
                    @pl.when(t + 1 < nch)
                    def _():
                        fetch(t + 1, 1 - bslot)

                    wait(bslot)

                    def row(i, _):
                        aepad_v[i, pl.ds(0, 16)] = aev2[bslot, i, :]
                        return 0

                    lax.fori_loop(0, BSC, row, 0)
                    pltpu.sync_copy(aepad_v,
                                    den_sh.at[idxd2.at[bslot]], add=True)
            return 0

        lax.fori_loop(0, 20, outer, 0)
        plsc.subcore_barrier()
        _striped(lambda r0, nr: pltpu.sync_copy(
            den_sh.at[pl.ds(r0, nr)],
            den_h.at[pl.ds(cid * N + r0, nr)]), sid)

    return body(ae, dst, zeros16)


B4 = 64
NCH4 = E // B4
CW4 = NCH4 // NS
C4TAIL = CW4 * NS


def _p4(vcat, ef32, ae, src, dst, zeros128):
    @functools.partial(
        pl.kernel,
        out_type=jax.ShapeDtypeStruct((2 * N, 128), jnp.float32),
        mesh=_sc_mesh(),
        compiler_params=_SC_PARAMS,
        scratch_types=[pltpu.VMEM((2, B4), jnp.int32),
                       pltpu.VMEM((2, B4), jnp.int32),
                       pltpu.VMEM((2, B4, 128), jnp.float32),
                       pltpu.VMEM((2, B4, 128), jnp.float32),
                       pltpu.VMEM((2, B4, 16), jnp.float32),
                       pltpu.VMEM_SHARED((N, 128), jnp.float32)]
        + [pltpu.SemaphoreType.DMA] * 10)
    def body(v_h, e_h, ae_h, src_h, dst_h, zeros_h, out_h,
             idxs2, idxd2, vv2, ev2, aev2, acc_sh, *sems):
        cid = lax.axis_index("c")
        sid = lax.axis_index("s")
        _striped(lambda r0, nr: pltpu.sync_copy(
            zeros_h.at[pl.ds(r0, nr)], acc_sh.at[pl.ds(r0, nr)]), sid)
        plsc.subcore_barrier()
        nch = CW4 + jnp.where(sid < NCH4 - C4TAIL, 1, 0)
        ssem = [sems[:3], sems[3:6]]
        isem = [sems[6:8], sems[8:10]]

        def cbase_of(t):
            return jnp.where(t < CW4, (sid * CW4 + t) * B4,
                             (C4TAIL + sid) * B4)

        def idx_fetch(t, b):
            cb = cbase_of(t)
            si, sd = isem[b]
            pltpu.async_copy(src_h.at[pl.ds(cb, B4)], idxs2.at[b], si)
            pltpu.async_copy(dst_h.at[pl.ds(cb, B4)], idxd2.at[b], sd)

        def idx_wait(b):
            si, sd = isem[b]
            pltpu.make_async_copy(src_h.at[pl.ds(0, B4)], idxs2.at[b],
                                  si).wait()
            pltpu.make_async_copy(dst_h.at[pl.ds(0, B4)], idxd2.at[b],
                                  sd).wait()

        def data_issue(t, b):
            cb = cbase_of(t)

            def shift(j, _):
                idxs2[b, pl.ds(j * 16, 16)] = (
                    idxs2[b, pl.ds(j * 16, 16)] + cid * N)
                return 0

            lax.fori_loop(0, B4 // 16, shift, 0)
            sv, se, sa = ssem[b]
            pltpu.async_copy(v_h.at[idxs2.at[b]], vv2.at[b], sv)
            pltpu.async_copy(e_h.at[pl.ds(cid * E + cb, B4)], ev2.at[b], se)
            pltpu.async_copy(ae_h.at[pl.ds(cb, B4)], aev2.at[b], sa)

        def data_wait(b):
            sv, se, sa = ssem[b]
            pltpu.make_async_copy(v_h.at[idxs2.at[b]], vv2.at[b], sv).wait()
            pltpu.make_async_copy(e_h.at[pl.ds(0, B4)], ev2.at[b], se).wait()
            pltpu.make_async_copy(ae_h.at[pl.ds(0, B4)], aev2.at[b],
                                  sa).wait()

        def compute(b):
            def row(i, _):
                a = aev2[b, i, :]
                lane = lax.iota(jnp.int32, 16)
                for j in range(8):
                    sel = jnp.where(lane == cid * 4 + j // 2, 1.0, 0.0)
                    ab = jnp.full((16,), jnp.sum(a * sel))
                    vv2[b, i, pl.ds(j * 16, 16)] = (
                        vv2[b, i, pl.ds(j * 16, 16)]
                        + ev2[b, i, pl.ds(j * 16, 16)]) * ab
                return 0

            lax.fori_loop(0, B4, row, 0)
            pltpu.sync_copy(vv2.at[b], acc_sh.at[idxd2.at[b]], add=True)

        idx_fetch(0, 0)
        idx_wait(0)
        data_issue(0, 0)

        @pl.when(nch > 1)
        def _():
            idx_fetch(1, 1)

        def outer(o, _):
            for bslot in range(2):
                t = o * 2 + bslot

                @pl.when(t < nch)
                def _():
                    @pl.when(t + 1 < nch)
                    def _():
                        idx_wait(1 - bslot)
                        data_issue(t + 1, 1 - bslot)

                    data_wait(bslot)
                    compute(bslot)

                    @pl.when(t + 2 < nch)
                    def _():
                        idx_fetch(t + 2, bslot)
            return 0

        lax.fori_loop(0, (CW4 + 2) // 2, outer, 0)
        plsc.subcore_barrier()
        _striped(lambda r0, nr: pltpu.sync_copy(
            acc_sh.at[pl.ds(r0, nr)],
            out_h.at[pl.ds(cid * N + r0, nr)]), sid)

    return body(vcat, ef32, ae, src, dst, zeros128)


def _p6(ab_tab, src, dst):
    @functools.partial(
        pl.kernel,
        out_type=jax.ShapeDtypeStruct((E, 32), jnp.float32),
        mesh=_sc_mesh(),
        compiler_params=_SC_PARAMS,
        scratch_types=[pltpu.VMEM((2, BSC), jnp.int32),
                       pltpu.VMEM((2, BSC), jnp.int32),
                       pltpu.VMEM((2, BSC, 128), jnp.float32),
                       pltpu.VMEM((2, BSC, 128), jnp.float32),
                       pltpu.VMEM((BSC, 32), jnp.float32)]
        + [pltpu.SemaphoreType.DMA] * 8)
    def body(ab_h, src_h, dst_h, out_h, idxs2, idxd2, av2, bv2, gv, *sems):
        cid = lax.axis_index("c")
        sid = lax.axis_index("s")
        wid = cid * NS + sid
        nch = 39 + jnp.where(wid < 2, 1, 0)
        ssem = [sems[:2], sems[2:4]]
        isem = [sems[4:6], sems[6:8]]

        def cbase_of(t):
            return jnp.where(t < 39, wid * EW + t * BSC, ETAIL + wid * BSC)

        def idx_fetch(t, b):
            cb = cbase_of(t)
            si, sd = isem[b]
            pltpu.async_copy(src_h.at[pl.ds(cb, BSC)], idxs2.at[b], si)
            pltpu.async_copy(dst_h.at[pl.ds(cb, BSC)], idxd2.at[b], sd)

        def idx_wait(b):
            si, sd = isem[b]
            pltpu.make_async_copy(src_h.at[pl.ds(0, BSC)], idxs2.at[b],
                                  si).wait()
            pltpu.make_async_copy(dst_h.at[pl.ds(0, BSC)], idxd2.at[b],
                                  sd).wait()

        def data_issue(b):
            sa, sb = ssem[b]
            pltpu.async_copy(ab_h.at[idxs2.at[b]], av2.at[b], sa)
            pltpu.async_copy(ab_h.at[idxd2.at[b]], bv2.at[b], sb)

        def data_wait(b):
            sa, sb = ssem[b]
            pltpu.make_async_copy(ab_h.at[idxs2.at[b]], av2.at[b], sa).wait()
            pltpu.make_async_copy(ab_h.at[idxd2.at[b]], bv2.at[b], sb).wait()

        def compute(t, b):
            def row(i, _):
                gv[i, pl.ds(0, 16)] = (av2[b, i, pl.ds(0, 16)]
                                       + bv2[b, i, pl.ds(32, 16)])
                gv[i, pl.ds(16, 16)] = (av2[b, i, pl.ds(16, 16)]
                                        + bv2[b, i, pl.ds(48, 16)])
                return 0

            lax.fori_loop(0, BSC, row, 0)
            pltpu.sync_copy(gv, out_h.at[pl.ds(cbase_of(t), BSC)])

        idx_fetch(0, 0)
        idx_wait(0)
        data_issue(0)

        @pl.when(nch > 1)
        def _():
            idx_fetch(1, 1)

        def outer(o, _):
            for bslot in range(2):
                t = o * 2 + bslot

                @pl.when(t < nch)
                def _():
                    @pl.when(t + 1 < nch)
                    def _():
                        idx_wait(1 - bslot)
                        data_issue(1 - bslot)

                    data_wait(bslot)
                    compute(t, bslot)

                    @pl.when(t + 2 < nch)
                    def _():
                        idx_fetch(t + 2, bslot)
            return 0

        lax.fori_loop(0, 20, outer, 0)

    return body(ab_tab, src, dst)


def _p1a_body(cnt0, cnt1, x, demb, wq, bq, wk, bk, wv, bv, wsk, bsk, g, b,
              x_in_o, q_o, k_o, v0_o, v1_o, hskip_o):
    deg = (cnt0[:, 0:1] + cnt1[:, 0:1]).astype(jnp.int32)
    deg = jnp.clip(deg, 0, MAXDEG - 1)
    iota = jax.lax.broadcasted_iota(jnp.int32, (deg.shape[0], MAXDEG), 1)
    onehot = (iota == deg).astype(jnp.float32)
    x_in = x[...] + jnp.dot(onehot, demb[...],
                            preferred_element_type=jnp.float32)
    h = _ln(x_in, g[...], b[...])
    q = jnp.dot(h, wq[...], preferred_element_type=jnp.float32) + bq[...]
    k = jnp.dot(h, wk[...], preferred_element_type=jnp.float32) + bk[...]
    v = jnp.dot(h, wv[...], preferred_element_type=jnp.float32) + bv[...]
    hs = jnp.dot(h, wsk[...], preferred_element_type=jnp.float32) + bsk[...]
    x_in_o[...] = x_in
    q_o[...] = q
    k_o[...] = k
    v0_o[...] = v[:, :128]
    v1_o[...] = v[:, 128:]
    hskip_o[...] = hs


def _p1a(cnt0, cnt1, x, demb, wq, bq, wk, bk, wv, bv, wsk, bsk, g, b):
    nb = pl.BlockSpec((BN, 256), lambda i: (i, 0))
    nh = pl.BlockSpec((BN, 128), lambda i: (i, 0))
    n16 = pl.BlockSpec((BN, 16), lambda i: (i, 0))
    full = lambda s: pl.BlockSpec(s, lambda i: (0,) * len(s))
    return pl.pallas_call(
        _p1a_body,
        grid=(N // BN,),
        in_specs=[nh, nh, nb, full((MAXDEG, 256)),
                  full((256, 256)), full((1, 256)),
                  full((256, 256)), full((1, 256)),
                  full((256, 256)), full((1, 256)),
                  full((256, 256)), full((1, 256)),
                  full((1, 256)), full((1, 256))],
        out_specs=[nb, nb, nb, nh, nh, nb],
        out_shape=[jax.ShapeDtypeStruct((N, 256), jnp.float32),
                   jax.ShapeDtypeStruct((N, 256), jnp.float32),
                   jax.ShapeDtypeStruct((N, 256), jnp.float32),
                   jax.ShapeDtypeStruct((N, 128), jnp.float32),
                   jax.ShapeDtypeStruct((N, 128), jnp.float32),
                   jax.ShapeDtypeStruct((N, 256), jnp.float32)],
        interpret=_INTERP,
    )(cnt0, cnt1, x, demb, wq, bq, wk, bk, wv, bv, wsk, bsk, g, b)


def _p1b_body(ea, we, ef_o):
    ef_o[...] = jnp.dot(ea[...], we[...], preferred_element_type=jnp.float32)


def _p1b(ea, we):
    nblk = E // BE
    return pl.pallas_call(
        _p1b_body,
        grid=(2, nblk),
        in_specs=[pl.BlockSpec((BE, EDIM), lambda j, i: (i, 0)),
                  pl.BlockSpec((EDIM, 128), lambda j, i: (0, j))],
        out_specs=pl.BlockSpec((BE, 128), lambda j, i: (j * nblk + i, 0)),
        out_shape=jax.ShapeDtypeStruct((2 * E, 128), jnp.float32),
        interpret=_INTERP,
    )(ea, we)


def _p5_body(a0, a1, den0, den1, x_in, hskip, g, b, w1, b1, w2, b2,
             eng, enb, wab, x_new_o, ab_o):
    bn = a0.shape[0]
    attn = jnp.concatenate([a0[...], a1[...]], axis=-1)
    dh = den0[:, 0:8] + den1[:, 0:8]
    dfull = jnp.reshape(jnp.broadcast_to(dh[:, :, None], (bn, 8, 32)),
                        (bn, 256))
    attn = attn / (dfull + 1e-16)
    x_res = x_in[...] + attn + hskip[...]
    hf = _ln(x_res, g[...], b[...])
    up = jnp.dot(hf, w1[...], preferred_element_type=jnp.float32) + b1[...]
    act = _gelu(up)
    ffn = jnp.dot(act, w2[...], preferred_element_type=jnp.float32) + b2[...]
    x_new = x_res + ffn
    xn = _ln(x_new, eng[...], enb[...])
    ab = jnp.dot(xn, wab[...], preferred_element_type=jnp.float32)
    x_new_o[...] = x_new
    ab_o[...] = jnp.concatenate(
        [ab, jnp.zeros((bn, 64), jnp.float32)], axis=-1)


def _p5(a0, a1, den0, den1, x_in, hskip, g, b, w1, b1, w2, b2, eng, enb,
        wab):
    nb = pl.BlockSpec((BN, 256), lambda i: (i, 0))
    nh = pl.BlockSpec((BN, 128), lambda i: (i, 0))
    n16 = pl.BlockSpec((BN, 16), lambda i: (i, 0))
    full = lambda s: pl.BlockSpec(s, lambda i: (0,) * len(s))
    return pl.pallas_call(
        _p5_body,
        grid=(N // BN,),
        in_specs=[nh, nh, nh, nh, nb, nb, full((1, 256)), full((1, 256)),
                  full((256, 1024)), full((1, 1024)),
                  full((1024, 256)), full((1, 256)),
                  full((1, 256)), full((1, 256)), full((256, 64))],
        out_specs=[nb, nh],
        out_shape=[jax.ShapeDtypeStruct((N, 256), jnp.float32),
                   jax.ShapeDtypeStruct((N, 128), jnp.float32)],
        interpret=_INTERP,
    )(a0, a1, den0, den1, x_in, hskip, g, b, w1, b1, w2, b2, eng, enb, wab)


def _p7_body(gsum, ea, wc, eb, eg, ebb, e_new_o):
    ct = jnp.dot(ea[...], wc[...], preferred_element_type=jnp.float32)
    raw = jax.nn.relu(gsum[...] + ct + eb[...])
    delta = raw[:, :EDIM]
    gate = raw[:, EDIM:]
    tmp = ea[...] + jax.nn.sigmoid(gate) * delta
    e_new_o[...] = _ln(tmp, eg[...], ebb[...])


def _p7(gsum, ea, wc, eb, eg, ebb):
    full = lambda s: pl.BlockSpec(s, lambda i: (0,) * len(s))
    return pl.pallas_call(
        _p7_body,
        grid=(E // BE,),
        in_specs=[pl.BlockSpec((BE, 32), lambda i: (i, 0)),
                  pl.BlockSpec((BE, EDIM), lambda i: (i, 0)),
                  full((EDIM, 32)), full((1, 32)),
                  full((1, EDIM)), full((1, EDIM))],
        out_specs=pl.BlockSpec((BE, EDIM), lambda i: (i, 0)),
        out_shape=jax.ShapeDtypeStruct((E, EDIM), jnp.float32),
        interpret=_INTERP,
    )(gsum, ea, wc, eb, eg, ebb)


def kernel(x, edge_index, edge_attr, norm_g, norm_b, deg_emb, Wq, bq, Wk, bk,
           Wv, bv, We, Wskip, bskip, W1, b1, W2, b2, eu_ng, eu_nb, eu_W,
           eu_b, eu_eg, eu_eb):
    src = edge_index[0]
    dst = edge_index[1]
    r2 = lambda v: v.reshape(1, -1)
    zeros128 = jnp.zeros((N, 128), jnp.float32)
    ones128 = jnp.ones((BSC, 128), jnp.float32)

    cnt = _p0(src, zeros128, ones128)
    cnt0, cnt1 = cnt[:N], cnt[N:]

    x_in, q, k, v0, v1, hskip = _p1a(
        cnt0, cnt1, x, deg_emb, Wq, r2(bq), Wk, r2(bk), Wv, r2(bv),
        Wskip, r2(bskip), r2(norm_g), r2(norm_b))
    ecat = _p1b(edge_attr, We)

    ae = _p2a(q, k, ecat, src, dst)
    den = _p2b(ae, dst, zeros128)

    vcat = jnp.concatenate([v0, v1], axis=0)
    attn = _p4(vcat, ecat, ae, src, dst, zeros128)
    a0, a1 = attn[:N], attn[N:]
    den0, den1 = den[:N], den[N:]

    wab = jnp.concatenate([eu_W[:256], eu_W[256:512]], axis=1)
    x_new, ab = _p5(a0, a1, den0, den1, x_in, hskip, r2(norm_g), r2(norm_b),
                    W1, r2(b1), W2, r2(b2), r2(eu_ng), r2(eu_nb), wab)

    gsum = _p6(ab, src, dst)

    e_new = _p7(gsum, edge_attr, eu_W[512:], r2(eu_b), r2(eu_eg), r2(eu_eb))
    return (x_new, e_new)


# speedup vs baseline: 1.0895x; 1.0895x over previous
import functools
import math

import jax
import jax.numpy as jnp
from jax import lax
from jax.experimental import pallas as pl
from jax.experimental.pallas import tpu as pltpu
from jax.experimental.pallas import tpu_sc as plsc

N = 10000
E = 160000
DIM = 256
EDIM = 16
H = 8
C = DIM // H
MAXDEG = 512

BN = 2000
BE = 2000

_INTERP = False

NC = 2
NS = 16
NW = NC * NS
BSC = 128
EW = 39 * BSC
ETAIL = NW * EW
RPW = 624
RTAIL = N - RPW * NS


def _sc_mesh():
    return plsc.VectorSubcoreMesh(core_axis_name="c", subcore_axis_name="s")


_SC_PARAMS = pltpu.CompilerParams(needs_layout_passes=False)


def _striped(copy_fn, sid):
    copy_fn(sid * RPW, RPW)

    @pl.when(sid == NS - 1)
    def _():
        copy_fn(RPW * NS, RTAIL)


def _edge_chunks(chunk_fn, wid):
    base = wid * EW
    lax.fori_loop(0, EW // BSC,
                  lambda t, _: (chunk_fn(base + t * BSC), 0)[1], 0)

    @pl.when(wid < 2)
    def _():
        chunk_fn(ETAIL + wid * BSC)


def _erf(x):
    s = jnp.sign(x)
    ax = jnp.abs(x)
    t = 1.0 / (1.0 + 0.3275911 * ax)
    poly = t * (0.254829592 + t * (-0.284496736 + t * (1.421413741
           + t * (-1.453152027 + t * 1.061405429))))
    return s * (1.0 - poly * jnp.exp(-ax * ax))


def _gelu(x):
    return 0.5 * x * (1.0 + _erf(x * 0.7071067811865476))


def _ln(x, g, b, eps=1e-5):
    mu = jnp.mean(x, axis=-1, keepdims=True)
    var = jnp.mean((x - mu) ** 2, axis=-1, keepdims=True)
    return (x - mu) * jax.lax.rsqrt(var + eps) * g + b


def _p0(src, zeros128, ones128):
    @functools.partial(
        pl.kernel,
        out_type=jax.ShapeDtypeStruct((2 * N, 128), jnp.float32),
        mesh=_sc_mesh(),
        compiler_params=_SC_PARAMS,
        scratch_types=[pltpu.VMEM((BSC,), jnp.int32),
                       pltpu.VMEM((BSC, 128), jnp.float32),
                       pltpu.VMEM_SHARED((N, 128), jnp.float32)])
    def body(src_h, zeros_h, ones_h, out_h, idx_v, ones_v, cnt_sh):
        cid = lax.axis_index("c")
        sid = lax.axis_index("s")
        wid = cid * NS + sid
        _striped(lambda r0, nr: pltpu.sync_copy(
            zeros_h.at[pl.ds(r0, nr)], cnt_sh.at[pl.ds(r0, nr)]), sid)
        pltpu.sync_copy(ones_h, ones_v)
        plsc.subcore_barrier()

        def chunk(cbase):
            pltpu.sync_copy(src_h.at[pl.ds(cbase, BSC)], idx_v)
            pltpu.sync_copy(ones_v, cnt_sh.at[idx_v], add=True)

        _edge_chunks(chunk, wid)
        plsc.subcore_barrier()
        _striped(lambda r0, nr: pltpu.sync_copy(
            cnt_sh.at[pl.ds(r0, nr)],
            out_h.at[pl.ds(cid * N + r0, nr)]), sid)

    return body(src, zeros128, ones128)


B2 = 64
NCH2 = E // B2
CW2 = NCH2 // NW
C2TAIL = CW2 * NW


def _p2a(q, k, ecat, src, dst):
    inv_sqrt_c = 1.0 / math.sqrt(float(C))

    @functools.partial(
        pl.kernel,
        out_type=jax.ShapeDtypeStruct((E, 16), jnp.float32),
        mesh=_sc_mesh(),
        compiler_params=_SC_PARAMS,
        scratch_types=[pltpu.VMEM((2, B2), jnp.int32),
                       pltpu.VMEM((2, B2), jnp.int32),
                       pltpu.VMEM((2, B2, 256), jnp.float32),
                       pltpu.VMEM((2, B2, 256), jnp.float32),
                       pltpu.VMEM((2, B2, 128), jnp.float32),
                       pltpu.VMEM((2, B2, 128), jnp.float32),
                       pltpu.VMEM((B2, 16), jnp.float32)]
        + [pltpu.SemaphoreType.DMA] * 12)
    def body(q_h, k_h, e_h, src_h, dst_h, ae_h,
             idxs2, idxd2, kv2, qv2, e0v2, e1v2, aev, *sems):
        cid = lax.axis_index("c")
        sid = lax.axis_index("s")
        wid = cid * NS + sid
        nch = CW2 + jnp.where(wid < NCH2 - C2TAIL, 1, 0)
        ssem = [sems[:4], sems[4:8]]
        isem = [sems[8:10], sems[10:12]]

        def cbase_of(t):
            return jnp.where(t < CW2, (wid * CW2 + t) * B2,
                             (C2TAIL + wid) * B2)

        def idx_fetch(t, b):
            cb = cbase_of(t)
            si, sd = isem[b]
            pltpu.async_copy(src_h.at[pl.ds(cb, B2)], idxs2.at[b], si)
            pltpu.async_copy(dst_h.at[pl.ds(cb, B2)], idxd2.at[b], sd)

        def idx_wait(b):
            si, sd = isem[b]
            pltpu.make_async_copy(src_h.at[pl.ds(0, B2)], idxs2.at[b],
                                  si).wait()
            pltpu.make_async_copy(dst_h.at[pl.ds(0, B2)], idxd2.at[b],
                                  sd).wait()

        def data_issue(t, b):
            cb = cbase_of(t)
            sk, sq, se0, se1 = ssem[b]
            pltpu.async_copy(k_h.at[idxs2.at[b]], kv2.at[b], sk)
            pltpu.async_copy(q_h.at[idxd2.at[b]], qv2.at[b], sq)
            pltpu.async_copy(e_h.at[pl.ds(cb, B2)], e0v2.at[b], se0)
            pltpu.async_copy(e_h.at[pl.ds(E + cb, B2)], e1v2.at[b], se1)

        def data_wait(b):
            sk, sq, se0, se1 = ssem[b]
            pltpu.make_async_copy(k_h.at[idxs2.at[b]], kv2.at[b], sk).wait()
            pltpu.make_async_copy(q_h.at[idxd2.at[b]], qv2.at[b], sq).wait()
            pltpu.make_async_copy(e_h.at[pl.ds(0, B2)], e0v2.at[b],
                                  se0).wait()
            pltpu.make_async_copy(e_h.at[pl.ds(0, B2)], e1v2.at[b],
                                  se1).wait()

        def compute(t, b):
            @plsc.parallel_loop(0, B2, unroll=4)
            def row(i):
                lane = lax.iota(jnp.int32, 16)
                alpha = jnp.zeros((16,), jnp.float32)
                for h in range(H):
                    ev2 = e0v2 if h < 4 else e1v2
                    eoff = (h % 4) * 32
                    k0 = (kv2[b, i, pl.ds(h * 32, 16)]
                          + ev2[b, i, pl.ds(eoff, 16)])
                    k1 = (kv2[b, i, pl.ds(h * 32 + 16, 16)]
                          + ev2[b, i, pl.ds(eoff + 16, 16)])
                    p = (k0 * qv2[b, i, pl.ds(h * 32, 16)]
                         + k1 * qv2[b, i, pl.ds(h * 32 + 16, 16)])
                    t2 = jnp.sum(p)
                    alpha = jnp.where(lane == h, t2, alpha)
                ae = jnp.exp(alpha * inv_sqrt_c)
                aev[i, :] = ae
            pltpu.sync_copy(aev, ae_h.at[pl.ds(cbase_of(t), B2)])

        idx_fetch(0, 0)
        idx_wait(0)
        data_issue(0, 0)

        @pl.when(nch > 1)
        def _():
            idx_fetch(1, 1)

        def outer(o, _):
            for bslot in range(2):
                t = o * 2 + bslot

                @pl.when(t < nch)
                def _():
                    @pl.when(t + 1 < nch)
                    def _():
                        idx_wait(1 - bslot)
                        data_issue(t + 1, 1 - bslot)

                    data_wait(bslot)
                    compute(t, bslot)

                    @pl.when(t + 2 < nch)
                    def _():
                        idx_fetch(t + 2, bslot)
            return 0

        lax.fori_loop(0, (CW2 + 2) // 2, outer, 0)

    return body(q, k, ecat, src, dst)


def _p2b(ae, dst, zeros16):
    @functools.partial(
        pl.kernel,
        out_type=jax.ShapeDtypeStruct((2 * N, 128), jnp.float32),
        mesh=_sc_mesh(),
        compiler_params=_SC_PARAMS,
        scratch_types=[pltpu.VMEM((2, BSC), jnp.int32),
                       pltpu.VMEM((2, BSC, 16), jnp.float32),
                       pltpu.VMEM((BSC, 128), jnp.float32),
                       pltpu.VMEM_SHARED((N, 128), jnp.float32)]
        + [pltpu.SemaphoreType.DMA] * 4)
    def body(ae_h, dst_h, zeros_h, den_h, idxd2, aev2, aepad_v, den_sh,
             *sems):
        cid = lax.axis_index("c")
        sid = lax.axis_index("s")
        wid = cid * NS + sid
        _striped(lambda r0, nr: pltpu.sync_copy(
            zeros_h.at[pl.ds(r0, nr)], den_sh.at[pl.ds(r0, nr)]), sid)
        plsc.subcore_barrier()
        nch = 39 + jnp.where(wid < 2, 1, 0)
        ssem = [sems[:2], sems[2:]]

        def cbase_of(t):
            return jnp.where(t < 39, wid * EW + t * BSC, ETAIL + wid * BSC)

        def fetch(t, b):
            cb = cbase_of(t)
            si, sa = ssem[b]
            pltpu.async_copy(dst_h.at[pl.ds(cb, BSC)], idxd2.at[b], si)
            pltpu.async_copy(ae_h.at[pl.ds(cb, BSC)], aev2.at[b], sa)

        def wait(b):
            si, sa = ssem[b]
            pltpu.make_async_copy(dst_h.at[pl.ds(0, BSC)], idxd2.at[b],
                                  si).wait()
            pltpu.make_async_copy(ae_h.at[pl.ds(0, BSC)], aev2.at[b],
                                  sa).wait()

        fetch(0, 0)

        def outer(o, _):
            for bslot in range(2):
                t = o * 2 + bslot

                @pl.when(t < nch)
                def _():
                    @pl.when(t + 1 < nch)
                    def _():
                        fetch(t + 1, 1 - bslot)

                    wait(bslot)

                    @plsc.parallel_loop(0, BSC, unroll=8)
                    def row(i):
                        aepad_v[i, pl.ds(0, 16)] = aev2[bslot, i, :]
                    pltpu.sync_copy(aepad_v,
                                    den_sh.at[idxd2.at[bslot]], add=True)
            return 0

        lax.fori_loop(0, 20, outer, 0)
        plsc.subcore_barrier()
        _striped(lambda r0, nr: pltpu.sync_copy(
            den_sh.at[pl.ds(r0, nr)],
            den_h.at[pl.ds(cid * N + r0, nr)]), sid)

    return body(ae, dst, zeros16)


B4 = 64
NCH4 = E // B4
CW4 = NCH4 // NS
C4TAIL = CW4 * NS


def _p4(vcat, ef32, ae, src, dst, zeros128):
    @functools.partial(
        pl.kernel,
        out_type=jax.ShapeDtypeStruct((2 * N, 128), jnp.float32),
        mesh=_sc_mesh(),
        compiler_params=_SC_PARAMS,
        scratch_types=[pltpu.VMEM((2, B4), jnp.int32),
                       pltpu.VMEM((2, B4), jnp.int32),
                       pltpu.VMEM((2, B4, 128), jnp.float32),
                       pltpu.VMEM((2, B4, 128), jnp.float32),
                       pltpu.VMEM((2, B4, 16), jnp.float32),
                       pltpu.VMEM_SHARED((N, 128), jnp.float32)]
        + [pltpu.SemaphoreType.DMA] * 10)
    def body(v_h, e_h, ae_h, src_h, dst_h, zeros_h, out_h,
             idxs2, idxd2, vv2, ev2, aev2, acc_sh, *sems):
        cid = lax.axis_index("c")
        sid = lax.axis_index("s")
        _striped(lambda r0, nr: pltpu.sync_copy(
            zeros_h.at[pl.ds(r0, nr)], acc_sh.at[pl.ds(r0, nr)]), sid)
        plsc.subcore_barrier()
        nch = CW4 + jnp.where(sid < NCH4 - C4TAIL, 1, 0)
        ssem = [sems[:3], sems[3:6]]
        isem = [sems[6:8], sems[8:10]]

        def cbase_of(t):
            return jnp.where(t < CW4, (sid * CW4 + t) * B4,
                             (C4TAIL + sid) * B4)

        def idx_fetch(t, b):
            cb = cbase_of(t)
            si, sd = isem[b]
            pltpu.async_copy(src_h.at[pl.ds(cb, B4)], idxs2.at[b], si)
            pltpu.async_copy(dst_h.at[pl.ds(cb, B4)], idxd2.at[b], sd)

        def idx_wait(b):
            si, sd = isem[b]
            pltpu.make_async_copy(src_h.at[pl.ds(0, B4)], idxs2.at[b],
                                  si).wait()
            pltpu.make_async_copy(dst_h.at[pl.ds(0, B4)], idxd2.at[b],
                                  sd).wait()

        def data_issue(t, b):
            cb = cbase_of(t)

            def shift(j, _):
                idxs2[b, pl.ds(j * 16, 16)] = (
                    idxs2[b, pl.ds(j * 16, 16)] + cid * N)
                return 0

            lax.fori_loop(0, B4 // 16, shift, 0)
            sv, se, sa = ssem[b]
            pltpu.async_copy(v_h.at[idxs2.at[b]], vv2.at[b], sv)
            pltpu.async_copy(e_h.at[pl.ds(cid * E + cb, B4)], ev2.at[b], se)
            pltpu.async_copy(ae_h.at[pl.ds(cb, B4)], aev2.at[b], sa)

        def data_wait(b):
            sv, se, sa = ssem[b]
            pltpu.make_async_copy(v_h.at[idxs2.at[b]], vv2.at[b], sv).wait()
            pltpu.make_async_copy(e_h.at[pl.ds(0, B4)], ev2.at[b], se).wait()
            pltpu.make_async_copy(ae_h.at[pl.ds(0, B4)], aev2.at[b],
                                  sa).wait()

        def compute(b):
            @plsc.parallel_loop(0, B4, unroll=4)
            def row(i):
                a = aev2[b, i, :]
                lane = lax.iota(jnp.int32, 16)
                for j in range(8):
                    sel = jnp.where(lane == cid * 4 + j // 2, 1.0, 0.0)
                    ab = jnp.full((16,), jnp.sum(a * sel))
                    vv2[b, i, pl.ds(j * 16, 16)] = (
                        vv2[b, i, pl.ds(j * 16, 16)]
                        + ev2[b, i, pl.ds(j * 16, 16)]) * ab
            pltpu.sync_copy(vv2.at[b], acc_sh.at[idxd2.at[b]], add=True)

        idx_fetch(0, 0)
        idx_wait(0)
        data_issue(0, 0)

        @pl.when(nch > 1)
        def _():
            idx_fetch(1, 1)

        def outer(o, _):
            for bslot in range(2):
                t = o * 2 + bslot

                @pl.when(t < nch)
                def _():
                    @pl.when(t + 1 < nch)
                    def _():
                        idx_wait(1 - bslot)
                        data_issue(t + 1, 1 - bslot)

                    data_wait(bslot)
                    compute(bslot)

                    @pl.when(t + 2 < nch)
                    def _():
                        idx_fetch(t + 2, bslot)
            return 0

        lax.fori_loop(0, (CW4 + 2) // 2, outer, 0)
        plsc.subcore_barrier()
        _striped(lambda r0, nr: pltpu.sync_copy(
            acc_sh.at[pl.ds(r0, nr)],
            out_h.at[pl.ds(cid * N + r0, nr)]), sid)

    return body(vcat, ef32, ae, src, dst, zeros128)


def _p6(ab_tab, src, dst):
    @functools.partial(
        pl.kernel,
        out_type=jax.ShapeDtypeStruct((E, 32), jnp.float32),
        mesh=_sc_mesh(),
        compiler_params=_SC_PARAMS,
        scratch_types=[pltpu.VMEM((2, BSC), jnp.int32),
                       pltpu.VMEM((2, BSC), jnp.int32),
                       pltpu.VMEM((2, BSC, 128), jnp.float32),
                       pltpu.VMEM((2, BSC, 128), jnp.float32),
                       pltpu.VMEM((BSC, 32), jnp.float32)]
        + [pltpu.SemaphoreType.DMA] * 8)
    def body(ab_h, src_h, dst_h, out_h, idxs2, idxd2, av2, bv2, gv, *sems):
        cid = lax.axis_index("c")
        sid = lax.axis_index("s")
        wid = cid * NS + sid
        nch = 39 + jnp.where(wid < 2, 1, 0)
        ssem = [sems[:2], sems[2:4]]
        isem = [sems[4:6], sems[6:8]]

        def cbase_of(t):
            return jnp.where(t < 39, wid * EW + t * BSC, ETAIL + wid * BSC)

        def idx_fetch(t, b):
            cb = cbase_of(t)
            si, sd = isem[b]
            pltpu.async_copy(src_h.at[pl.ds(cb, BSC)], idxs2.at[b], si)
            pltpu.async_copy(dst_h.at[pl.ds(cb, BSC)], idxd2.at[b], sd)

        def idx_wait(b):
            si, sd = isem[b]
            pltpu.make_async_copy(src_h.at[pl.ds(0, BSC)], idxs2.at[b],
                                  si).wait()
            pltpu.make_async_copy(dst_h.at[pl.ds(0, BSC)], idxd2.at[b],
                                  sd).wait()

        def data_issue(b):
            sa, sb = ssem[b]
            pltpu.async_copy(ab_h.at[idxs2.at[b]], av2.at[b], sa)
            pltpu.async_copy(ab_h.at[idxd2.at[b]], bv2.at[b], sb)

        def data_wait(b):
            sa, sb = ssem[b]
            pltpu.make_async_copy(ab_h.at[idxs2.at[b]], av2.at[b], sa).wait()
            pltpu.make_async_copy(ab_h.at[idxd2.at[b]], bv2.at[b], sb).wait()

        def compute(t, b):
            @plsc.parallel_loop(0, BSC, unroll=8)
            def row(i):
                gv[i, pl.ds(0, 16)] = (av2[b, i, pl.ds(0, 16)]
                                       + bv2[b, i, pl.ds(32, 16)])
                gv[i, pl.ds(16, 16)] = (av2[b, i, pl.ds(16, 16)]
                                        + bv2[b, i, pl.ds(48, 16)])
            pltpu.sync_copy(gv, out_h.at[pl.ds(cbase_of(t), BSC)])

        idx_fetch(0, 0)
        idx_wait(0)
        data_issue(0)

        @pl.when(nch > 1)
        def _():
            idx_fetch(1, 1)

        def outer(o, _):
            for bslot in range(2):
                t = o * 2 + bslot

                @pl.when(t < nch)
                def _():
                    @pl.when(t + 1 < nch)
                    def _():
                        idx_wait(1 - bslot)
                        data_issue(1 - bslot)

                    data_wait(bslot)
                    compute(t, bslot)

                    @pl.when(t + 2 < nch)
                    def _():
                        idx_fetch(t + 2, bslot)
            return 0

        lax.fori_loop(0, 20, outer, 0)

    return body(ab_tab, src, dst)


def _p1a_body(cnt0, cnt1, x, demb, wq, bq, wk, bk, wv, bv, wsk, bsk, g, b,
              x_in_o, q_o, k_o, v0_o, v1_o, hskip_o):
    deg = (cnt0[:, 0:1] + cnt1[:, 0:1]).astype(jnp.int32)
    deg = jnp.clip(deg, 0, MAXDEG - 1)
    iota = jax.lax.broadcasted_iota(jnp.int32, (deg.shape[0], MAXDEG), 1)
    onehot = (iota == deg).astype(jnp.float32)
    x_in = x[...] + jnp.dot(onehot, demb[...],
                            preferred_element_type=jnp.float32)
    h = _ln(x_in, g[...], b[...])
    q = jnp.dot(h, wq[...], preferred_element_type=jnp.float32) + bq[...]
    k = jnp.dot(h, wk[...], preferred_element_type=jnp.float32) + bk[...]
    v = jnp.dot(h, wv[...], preferred_element_type=jnp.float32) + bv[...]
    hs = jnp.dot(h, wsk[...], preferred_element_type=jnp.float32) + bsk[...]
    x_in_o[...] = x_in
    q_o[...] = q
    k_o[...] = k
    v0_o[...] = v[:, :128]
    v1_o[...] = v[:, 128:]
    hskip_o[...] = hs


def _p1a(cnt0, cnt1, x, demb, wq, bq, wk, bk, wv, bv, wsk, bsk, g, b):
    nb = pl.BlockSpec((BN, 256), lambda i: (i, 0))
    nh = pl.BlockSpec((BN, 128), lambda i: (i, 0))
    n16 = pl.BlockSpec((BN, 16), lambda i: (i, 0))
    full = lambda s: pl.BlockSpec(s, lambda i: (0,) * len(s))
    return pl.pallas_call(
        _p1a_body,
        grid=(N // BN,),
        in_specs=[pl.BlockSpec((BN, 128), lambda i: (i, 0)),
                  pl.BlockSpec((BN, 128), lambda i: (N // BN + i, 0)),
                  nb, full((MAXDEG, 256)),
                  full((256, 256)), full((1, 256)),
                  full((256, 256)), full((1, 256)),
                  full((256, 256)), full((1, 256)),
                  full((256, 256)), full((1, 256)),
                  full((1, 256)), full((1, 256))],
        out_specs=[nb, nb, nb, nh, nh, nb],
        out_shape=[jax.ShapeDtypeStruct((N, 256), jnp.float32),
                   jax.ShapeDtypeStruct((N, 256), jnp.float32),
                   jax.ShapeDtypeStruct((N, 256), jnp.float32),
                   jax.ShapeDtypeStruct((N, 128), jnp.float32),
                   jax.ShapeDtypeStruct((N, 128), jnp.float32),
                   jax.ShapeDtypeStruct((N, 256), jnp.float32)],
        interpret=_INTERP,
    )(cnt0, cnt1, x, demb, wq, bq, wk, bk, wv, bv, wsk, bsk, g, b)


def _p1b_body(ea, we, ef_o):
    ef_o[...] = jnp.dot(ea[...], we[...], preferred_element_type=jnp.float32)


def _p1b(ea, we):
    nblk = E // BE
    return pl.pallas_call(
        _p1b_body,
        grid=(2, nblk),
        in_specs=[pl.BlockSpec((BE, EDIM), lambda j, i: (i, 0)),
                  pl.BlockSpec((EDIM, 128), lambda j, i: (0, j))],
        out_specs=pl.BlockSpec((BE, 128), lambda j, i: (j * nblk + i, 0)),
        out_shape=jax.ShapeDtypeStruct((2 * E, 128), jnp.float32),
        interpret=_INTERP,
    )(ea, we)


def _p5_body(a0, a1, den0, den1, x_in, hskip, g, b, w1, b1, w2, b2,
             eng, enb, wab, x_new_o, ab_o):
    bn = a0.shape[0]
    attn = jnp.concatenate([a0[...], a1[...]], axis=-1)
    dh = den0[:, 0:8] + den1[:, 0:8]
    dfull = jnp.reshape(jnp.broadcast_to(dh[:, :, None], (bn, 8, 32)),
                        (bn, 256))
    attn = attn / (dfull + 1e-16)
    x_res = x_in[...] + attn + hskip[...]
    hf = _ln(x_res, g[...], b[...])
    up = jnp.dot(hf, w1[...], preferred_element_type=jnp.float32) + b1[...]
    act = _gelu(up)
    ffn = jnp.dot(act, w2[...], preferred_element_type=jnp.float32) + b2[...]
    x_new = x_res + ffn
    xn = _ln(x_new, eng[...], enb[...])
    ab = jnp.dot(xn, wab[...], preferred_element_type=jnp.float32)
    x_new_o[...] = x_new
    ab_o[...] = jnp.concatenate(
        [ab, jnp.zeros((bn, 64), jnp.float32)], axis=-1)


def _p5(a0, a1, den0, den1, x_in, hskip, g, b, w1, b1, w2, b2, eng, enb,
        wab):
    nb = pl.BlockSpec((BN, 256), lambda i: (i, 0))
    nh = pl.BlockSpec((BN, 128), lambda i: (i, 0))
    n16 = pl.BlockSpec((BN, 16), lambda i: (i, 0))
    full = lambda s: pl.BlockSpec(s, lambda i: (0,) * len(s))
    return pl.pallas_call(
        _p5_body,
        grid=(N // BN,),
        in_specs=[pl.BlockSpec((BN, 128), lambda i: (i, 0)),
                  pl.BlockSpec((BN, 128), lambda i: (N // BN + i, 0)),
                  pl.BlockSpec((BN, 128), lambda i: (i, 0)),
                  pl.BlockSpec((BN, 128), lambda i: (N // BN + i, 0)),
                  nb, nb, full((1, 256)), full((1, 256)),
                  full((256, 1024)), full((1, 1024)),
                  full((1024, 256)), full((1, 256)),
                  full((1, 256)), full((1, 256)), full((256, 64))],
        out_specs=[nb, nh],
        out_shape=[jax.ShapeDtypeStruct((N, 256), jnp.float32),
                   jax.ShapeDtypeStruct((N, 128), jnp.float32)],
        interpret=_INTERP,
    )(a0, a1, den0, den1, x_in, hskip, g, b, w1, b1, w2, b2, eng, enb, wab)


def _p7_body(gsum, ea, wc, eb, eg, ebb, e_new_o):
    ct = jnp.dot(ea[...], wc[...], preferred_element_type=jnp.float32)
    raw = jax.nn.relu(gsum[...] + ct + eb[...])
    delta = raw[:, :EDIM]
    gate = raw[:, EDIM:]
    tmp = ea[...] + jax.nn.sigmoid(gate) * delta
    e_new_o[...] = _ln(tmp, eg[...], ebb[...])


def _p7(gsum, ea, wc, eb, eg, ebb):
    full = lambda s: pl.BlockSpec(s, lambda i: (0,) * len(s))
    return pl.pallas_call(
        _p7_body,
        grid=(E // BE,),
        in_specs=[pl.BlockSpec((BE, 32), lambda i: (i, 0)),
                  pl.BlockSpec((BE, EDIM), lambda i: (i, 0)),
                  full((EDIM, 32)), full((1, 32)),
                  full((1, EDIM)), full((1, EDIM))],
        out_specs=pl.BlockSpec((BE, EDIM), lambda i: (i, 0)),
        out_shape=jax.ShapeDtypeStruct((E, EDIM), jnp.float32),
        interpret=_INTERP,
    )(gsum, ea, wc, eb, eg, ebb)


def kernel(x, edge_index, edge_attr, norm_g, norm_b, deg_emb, Wq, bq, Wk, bk,
           Wv, bv, We, Wskip, bskip, W1, b1, W2, b2, eu_ng, eu_nb, eu_W,
           eu_b, eu_eg, eu_eb):
    src = edge_index[0]
    dst = edge_index[1]
    r2 = lambda v: v.reshape(1, -1)
    zeros128 = jnp.zeros((N, 128), jnp.float32)
    ones128 = jnp.ones((BSC, 128), jnp.float32)

    cnt = _p0(src, zeros128, ones128)

    x_in, q, k, v0, v1, hskip = _p1a(
        cnt, cnt, x, deg_emb, Wq, r2(bq), Wk, r2(bk), Wv, r2(bv),
        Wskip, r2(bskip), r2(norm_g), r2(norm_b))
    ecat = _p1b(edge_attr, We)

    ae = _p2a(q, k, ecat, src, dst)
    den = _p2b(ae, dst, zeros128)

    vcat = jnp.concatenate([v0, v1], axis=0)
    attn = _p4(vcat, ecat, ae, src, dst, zeros128)

    wab = jnp.concatenate([eu_W[:256], eu_W[256:512]], axis=1)
    x_new, ab = _p5(attn, attn, den, den, x_in, hskip, r2(norm_g), r2(norm_b),
                    W1, r2(b1), W2, r2(b2), r2(eu_ng), r2(eu_nb), wab)

    gsum = _p6(ab, src, dst)

    e_new = _p7(gsum, edge_attr, eu_W[512:], r2(eu_b), r2(eu_eg), r2(eu_eb))
    return (x_new, e_new)


# speedup vs baseline: 1.1910x; 1.0932x over previous
import functools
import math

import jax
import jax.numpy as jnp
from jax import lax
from jax.experimental import pallas as pl
from jax.experimental.pallas import tpu as pltpu
from jax.experimental.pallas import tpu_sc as plsc

N = 10000
E = 160000
DIM = 256
EDIM = 16
H = 8
C = DIM // H
MAXDEG = 512

BN = 2000
BE = 8000

_INTERP = False

NC = 2
NS = 16
NW = NC * NS
BSC = 128
EW = 39 * BSC
ETAIL = NW * EW
RPW = 624
RTAIL = N - RPW * NS


def _sc_mesh():
    return plsc.VectorSubcoreMesh(core_axis_name="c", subcore_axis_name="s")


_SC_PARAMS = pltpu.CompilerParams(needs_layout_passes=False)


def _striped(copy_fn, sid):
    copy_fn(sid * RPW, RPW)

    @pl.when(sid == NS - 1)
    def _():
        copy_fn(RPW * NS, RTAIL)


def _edge_chunks(chunk_fn, wid):
    base = wid * EW
    lax.fori_loop(0, EW // BSC,
                  lambda t, _: (chunk_fn(base + t * BSC), 0)[1], 0)

    @pl.when(wid < 2)
    def _():
        chunk_fn(ETAIL + wid * BSC)


def _erf(x):
    s = jnp.sign(x)
    ax = jnp.abs(x)
    t = 1.0 / (1.0 + 0.3275911 * ax)
    poly = t * (0.254829592 + t * (-0.284496736 + t * (1.421413741
           + t * (-1.453152027 + t * 1.061405429))))
    return s * (1.0 - poly * jnp.exp(-ax * ax))


def _gelu(x):
    return 0.5 * x * (1.0 + _erf(x * 0.7071067811865476))


def _ln(x, g, b, eps=1e-5):
    mu = jnp.mean(x, axis=-1, keepdims=True)
    var = jnp.mean((x - mu) ** 2, axis=-1, keepdims=True)
    return (x - mu) * jax.lax.rsqrt(var + eps) * g + b


def _p0(src, zeros128, ones128):
    @functools.partial(
        pl.kernel,
        out_type=jax.ShapeDtypeStruct((2 * N, 128), jnp.float32),
        mesh=_sc_mesh(),
        compiler_params=_SC_PARAMS,
        scratch_types=[pltpu.VMEM((BSC,), jnp.int32),
                       pltpu.VMEM((BSC, 128), jnp.float32),
                       pltpu.VMEM_SHARED((N, 128), jnp.float32)])
    def body(src_h, zeros_h, ones_h, out_h, idx_v, ones_v, cnt_sh):
        cid = lax.axis_index("c")
        sid = lax.axis_index("s")
        wid = cid * NS + sid
        _striped(lambda r0, nr: pltpu.sync_copy(
            zeros_h.at[pl.ds(r0, nr)], cnt_sh.at[pl.ds(r0, nr)]), sid)
        pltpu.sync_copy(ones_h, ones_v)
        plsc.subcore_barrier()

        def chunk(cbase):
            pltpu.sync_copy(src_h.at[pl.ds(cbase, BSC)], idx_v)
            pltpu.sync_copy(ones_v, cnt_sh.at[idx_v], add=True)

        _edge_chunks(chunk, wid)
        plsc.subcore_barrier()
        _striped(lambda r0, nr: pltpu.sync_copy(
            cnt_sh.at[pl.ds(r0, nr)],
            out_h.at[pl.ds(cid * N + r0, nr)]), sid)

    return body(src, zeros128, ones128)


B2 = 64
NCH2 = E // B2
CW2 = NCH2 // NW
C2TAIL = CW2 * NW


def _p2a(q, k, ecat, src, dst):
    inv_sqrt_c = 1.0 / math.sqrt(float(C))

    @functools.partial(
        pl.kernel,
        out_type=jax.ShapeDtypeStruct((E, 16), jnp.float32),
        mesh=_sc_mesh(),
        compiler_params=_SC_PARAMS,
        scratch_types=[pltpu.VMEM((2, B2), jnp.int32),
                       pltpu.VMEM((2, B2), jnp.int32),
                       pltpu.VMEM((2, B2, 256), jnp.float32),
                       pltpu.VMEM((2, B2, 256), jnp.float32),
                       pltpu.VMEM((2, B2, 128), jnp.float32),
                       pltpu.VMEM((2, B2, 128), jnp.float32),
                       pltpu.VMEM((B2, 16), jnp.float32)]
        + [pltpu.SemaphoreType.DMA] * 12)
    def body(q_h, k_h, e_h, src_h, dst_h, ae_h,
             idxs2, idxd2, kv2, qv2, e0v2, e1v2, aev, *sems):
        cid = lax.axis_index("c")
        sid = lax.axis_index("s")
        wid = cid * NS + sid
        nch = CW2 + jnp.where(wid < NCH2 - C2TAIL, 1, 0)
        ssem = [sems[:4], sems[4:8]]
        isem = [sems[8:10], sems[10:12]]

        def cbase_of(t):
            return jnp.where(t < CW2, (wid * CW2 + t) * B2,
                             (C2TAIL + wid) * B2)

        def idx_fetch(t, b):
            cb = cbase_of(t)
            si, sd = isem[b]
            pltpu.async_copy(src_h.at[pl.ds(cb, B2)], idxs2.at[b], si)
            pltpu.async_copy(dst_h.at[pl.ds(cb, B2)], idxd2.at[b], sd)

        def idx_wait(b):
            si, sd = isem[b]
            pltpu.make_async_copy(src_h.at[pl.ds(0, B2)], idxs2.at[b],
                                  si).wait()
            pltpu.make_async_copy(dst_h.at[pl.ds(0, B2)], idxd2.at[b],
                                  sd).wait()

        def data_issue(t, b):
            cb = cbase_of(t)
            sk, sq, se0, se1 = ssem[b]
            pltpu.async_copy(k_h.at[idxs2.at[b]], kv2.at[b], sk)
            pltpu.async_copy(q_h.at[idxd2.at[b]], qv2.at[b], sq)
            pltpu.async_copy(e_h.at[pl.ds(cb, B2)], e0v2.at[b], se0)
            pltpu.async_copy(e_h.at[pl.ds(E + cb, B2)], e1v2.at[b], se1)

        def data_wait(b):
            sk, sq, se0, se1 = ssem[b]
            pltpu.make_async_copy(k_h.at[idxs2.at[b]], kv2.at[b], sk).wait()
            pltpu.make_async_copy(q_h.at[idxd2.at[b]], qv2.at[b], sq).wait()
            pltpu.make_async_copy(e_h.at[pl.ds(0, B2)], e0v2.at[b],
                                  se0).wait()
            pltpu.make_async_copy(e_h.at[pl.ds(0, B2)], e1v2.at[b],
                                  se1).wait()

        def compute(t, b):
            @plsc.parallel_loop(0, B2, unroll=4)
            def row(i):
                lane = lax.iota(jnp.int32, 16)
                alpha = jnp.zeros((16,), jnp.float32)
                for h in range(H):
                    ev2 = e0v2 if h < 4 else e1v2
                    eoff = (h % 4) * 32
                    k0 = (kv2[b, i, pl.ds(h * 32, 16)]
                          + ev2[b, i, pl.ds(eoff, 16)])
                    k1 = (kv2[b, i, pl.ds(h * 32 + 16, 16)]
                          + ev2[b, i, pl.ds(eoff + 16, 16)])
                    p = (k0 * qv2[b, i, pl.ds(h * 32, 16)]
                         + k1 * qv2[b, i, pl.ds(h * 32 + 16, 16)])
                    t2 = jnp.sum(p)
                    alpha = jnp.where(lane == h, t2, alpha)
                ae = jnp.exp(alpha * inv_sqrt_c)
                aev[i, :] = ae
            pltpu.sync_copy(aev, ae_h.at[pl.ds(cbase_of(t), B2)])

        idx_fetch(0, 0)
        idx_wait(0)
        data_issue(0, 0)

        @pl.when(nch > 1)
        def _():
            idx_fetch(1, 1)

        def outer(o, _):
            for bslot in range(2):
                t = o * 2 + bslot

                @pl.when(t < nch)
                def _():
                    @pl.when(t + 1 < nch)
                    def _():
                        idx_wait(1 - bslot)
                        data_issue(t + 1, 1 - bslot)

                    data_wait(bslot)
                    compute(t, bslot)

                    @pl.when(t + 2 < nch)
                    def _():
                        idx_fetch(t + 2, bslot)
            return 0

        lax.fori_loop(0, (CW2 + 2) // 2, outer, 0)

    return body(q, k, ecat, src, dst)


def _p2b(ae, dst, zeros16):
    @functools.partial(
        pl.kernel,
        out_type=jax.ShapeDtypeStruct((2 * N, 128), jnp.float32),
        mesh=_sc_mesh(),
        compiler_params=_SC_PARAMS,
        scratch_types=[pltpu.VMEM((2, BSC), jnp.int32),
                       pltpu.VMEM((2, BSC, 16), jnp.float32),
                       pltpu.VMEM((BSC, 128), jnp.float32),
                       pltpu.VMEM_SHARED((N, 128), jnp.float32)]
        + [pltpu.SemaphoreType.DMA] * 4)
    def body(ae_h, dst_h, zeros_h, den_h, idxd2, aev2, aepad_v, den_sh,
             *sems):
        cid = lax.axis_index("c")
        sid = lax.axis_index("s")
        wid = cid * NS + sid
        _striped(lambda r0, nr: pltpu.sync_copy(
            zeros_h.at[pl.ds(r0, nr)], den_sh.at[pl.ds(r0, nr)]), sid)
        plsc.subcore_barrier()
        nch = 39 + jnp.where(wid < 2, 1, 0)
        ssem = [sems[:2], sems[2:]]

        def cbase_of(t):
            return jnp.where(t < 39, wid * EW + t * BSC, ETAIL + wid * BSC)

        def fetch(t, b):
            cb = cbase_of(t)
            si, sa = ssem[b]
            pltpu.async_copy(dst_h.at[pl.ds(cb, BSC)], idxd2.at[b], si)
            pltpu.async_copy(ae_h.at[pl.ds(cb, BSC)], aev2.at[b], sa)

        def wait(b):
            si, sa = ssem[b]
            pltpu.make_async_copy(dst_h.at[pl.ds(0, BSC)], idxd2.at[b],
                                  si).wait()
            pltpu.make_async_copy(ae_h.at[pl.ds(0, BSC)], aev2.at[b],
                                  sa).wait()

        fetch(0, 0)

        def outer(o, _):
            for bslot in range(2):
                t = o * 2 + bslot

                @pl.when(t < nch)
                def _():
                    @pl.when(t + 1 < nch)
                    def _():
                        fetch(t + 1, 1 - bslot)

                    wait(bslot)

                    @plsc.parallel_loop(0, BSC, unroll=8)
                    def row(i):
                        aepad_v[i, pl.ds(0, 16)] = aev2[bslot, i, :]
                    pltpu.sync_copy(aepad_v,
                                    den_sh.at[idxd2.at[bslot]], add=True)
            return 0

        lax.fori_loop(0, 20, outer, 0)
        plsc.subcore_barrier()
        _striped(lambda r0, nr: pltpu.sync_copy(
            den_sh.at[pl.ds(r0, nr)],
            den_h.at[pl.ds(cid * N + r0, nr)]), sid)

    return body(ae, dst, zeros16)


B4 = 64
NCH4 = E // B4
CW4 = NCH4 // NS
C4TAIL = CW4 * NS


def _p4(v0t, v1t, ef32, ae, src, dst, zeros128):
    @functools.partial(
        pl.kernel,
        out_type=jax.ShapeDtypeStruct((2 * N, 128), jnp.float32),
        mesh=_sc_mesh(),
        compiler_params=_SC_PARAMS,
        scratch_types=[pltpu.VMEM((2, B4), jnp.int32),
                       pltpu.VMEM((2, B4), jnp.int32),
                       pltpu.VMEM((2, B4, 128), jnp.float32),
                       pltpu.VMEM((2, B4, 128), jnp.float32),
                       pltpu.VMEM((2, B4, 16), jnp.float32),
                       pltpu.VMEM_SHARED((N, 128), jnp.float32)]
        + [pltpu.SemaphoreType.DMA] * 10)
    def body(v0_h, v1_h, e_h, ae_h, src_h, dst_h, zeros_h, out_h,
             idxs2, idxd2, vv2, ev2, aev2, acc_sh, *sems):
        cid = lax.axis_index("c")
        sid = lax.axis_index("s")
        _striped(lambda r0, nr: pltpu.sync_copy(
            zeros_h.at[pl.ds(r0, nr)], acc_sh.at[pl.ds(r0, nr)]), sid)
        plsc.subcore_barrier()
        nch = CW4 + jnp.where(sid < NCH4 - C4TAIL, 1, 0)
        ssem = [sems[:3], sems[3:6]]
        isem = [sems[6:8], sems[8:10]]

        def cbase_of(t):
            return jnp.where(t < CW4, (sid * CW4 + t) * B4,
                             (C4TAIL + sid) * B4)

        def idx_fetch(t, b):
            cb = cbase_of(t)
            si, sd = isem[b]
            pltpu.async_copy(src_h.at[pl.ds(cb, B4)], idxs2.at[b], si)
            pltpu.async_copy(dst_h.at[pl.ds(cb, B4)], idxd2.at[b], sd)

        def idx_wait(b):
            si, sd = isem[b]
            pltpu.make_async_copy(src_h.at[pl.ds(0, B4)], idxs2.at[b],
                                  si).wait()
            pltpu.make_async_copy(dst_h.at[pl.ds(0, B4)], idxd2.at[b],
                                  sd).wait()

        def data_issue(t, b):
            cb = cbase_of(t)
            sv, se, sa = ssem[b]

            @pl.when(cid == 0)
            def _():
                pltpu.async_copy(v0_h.at[idxs2.at[b]], vv2.at[b], sv)

            @pl.when(cid == 1)
            def _():
                pltpu.async_copy(v1_h.at[idxs2.at[b]], vv2.at[b], sv)

            pltpu.async_copy(e_h.at[pl.ds(cid * E + cb, B4)], ev2.at[b], se)
            pltpu.async_copy(ae_h.at[pl.ds(cb, B4)], aev2.at[b], sa)

        def data_wait(b):
            sv, se, sa = ssem[b]
            pltpu.make_async_copy(v0_h.at[idxs2.at[b]], vv2.at[b], sv).wait()
            pltpu.make_async_copy(e_h.at[pl.ds(0, B4)], ev2.at[b], se).wait()
            pltpu.make_async_copy(ae_h.at[pl.ds(0, B4)], aev2.at[b],
                                  sa).wait()

        def compute(b):
            @plsc.parallel_loop(0, B4, unroll=4)
            def row(i):
                a = aev2[b, i, :]
                lane = lax.iota(jnp.int32, 16)
                abh = []
                for hh in range(4):
                    sel = jnp.where(lane == cid * 4 + hh, 1.0, 0.0)
                    abh.append(jnp.full((16,), jnp.sum(a * sel)))
                for j in range(8):
                    vv2[b, i, pl.ds(j * 16, 16)] = (
                        vv2[b, i, pl.ds(j * 16, 16)]
                        + ev2[b, i, pl.ds(j * 16, 16)]) * abh[j // 2]
            pltpu.sync_copy(vv2.at[b], acc_sh.at[idxd2.at[b]], add=True)

        idx_fetch(0, 0)
        idx_wait(0)
        data_issue(0, 0)

        @pl.when(nch > 1)
        def _():
            idx_fetch(1, 1)

        def outer(o, _):
            for bslot in range(2):
                t = o * 2 + bslot

                @pl.when(t < nch)
                def _():
                    @pl.when(t + 1 < nch)
                    def _():
                        idx_wait(1 - bslot)
                        data_issue(t + 1, 1 - bslot)

                    data_wait(bslot)
                    compute(bslot)

                    @pl.when(t + 2 < nch)
                    def _():
                        idx_fetch(t + 2, bslot)
            return 0

        lax.fori_loop(0, (CW4 + 2) // 2, outer, 0)
        plsc.subcore_barrier()
        _striped(lambda r0, nr: pltpu.sync_copy(
            acc_sh.at[pl.ds(r0, nr)],
            out_h.at[pl.ds(cid * N + r0, nr)]), sid)

    return body(v0t, v1t, ef32, ae, src, dst, zeros128)


def _p6(ab_tab, src, dst):
    @functools.partial(
        pl.kernel,
        out_type=jax.ShapeDtypeStruct((E, 32), jnp.float32),
        mesh=_sc_mesh(),
        compiler_params=_SC_PARAMS,
        scratch_types=[pltpu.VMEM((2, BSC), jnp.int32),
                       pltpu.VMEM((2, BSC), jnp.int32),
                       pltpu.VMEM((2, BSC, 128), jnp.float32),
                       pltpu.VMEM((2, BSC, 128), jnp.float32),
                       pltpu.VMEM((BSC, 32), jnp.float32)]
        + [pltpu.SemaphoreType.DMA] * 8)
    def body(ab_h, src_h, dst_h, out_h, idxs2, idxd2, av2, bv2, gv, *sems):
        cid = lax.axis_index("c")
        sid = lax.axis_index("s")
        wid = cid * NS + sid
        nch = 39 + jnp.where(wid < 2, 1, 0)
        ssem = [sems[:2], sems[2:4]]
        isem = [sems[4:6], sems[6:8]]

        def cbase_of(t):
            return jnp.where(t < 39, wid * EW + t * BSC, ETAIL + wid * BSC)

        def idx_fetch(t, b):
            cb = cbase_of(t)
            si, sd = isem[b]
            pltpu.async_copy(src_h.at[pl.ds(cb, BSC)], idxs2.at[b], si)
            pltpu.async_copy(dst_h.at[pl.ds(cb, BSC)], idxd2.at[b], sd)

        def idx_wait(b):
            si, sd = isem[b]
            pltpu.make_async_copy(src_h.at[pl.ds(0, BSC)], idxs2.at[b],
                                  si).wait()
            pltpu.make_async_copy(dst_h.at[pl.ds(0, BSC)], idxd2.at[b],
                                  sd).wait()

        def data_issue(b):
            sa, sb = ssem[b]
            pltpu.async_copy(ab_h.at[idxs2.at[b]], av2.at[b], sa)
            pltpu.async_copy(ab_h.at[idxd2.at[b]], bv2.at[b], sb)

        def data_wait(b):
            sa, sb = ssem[b]
            pltpu.make_async_copy(ab_h.at[idxs2.at[b]], av2.at[b], sa).wait()
            pltpu.make_async_copy(ab_h.at[idxd2.at[b]], bv2.at[b], sb).wait()

        def compute(t, b):
            @plsc.parallel_loop(0, BSC, unroll=8)
            def row(i):
                gv[i, pl.ds(0, 16)] = (av2[b, i, pl.ds(0, 16)]
                                       + bv2[b, i, pl.ds(32, 16)])
                gv[i, pl.ds(16, 16)] = (av2[b, i, pl.ds(16, 16)]
                                        + bv2[b, i, pl.ds(48, 16)])
            pltpu.sync_copy(gv, out_h.at[pl.ds(cbase_of(t), BSC)])

        idx_fetch(0, 0)
        idx_wait(0)
        data_issue(0)

        @pl.when(nch > 1)
        def _():
            idx_fetch(1, 1)

        def outer(o, _):
            for bslot in range(2):
                t = o * 2 + bslot

                @pl.when(t < nch)
                def _():
                    @pl.when(t + 1 < nch)
                    def _():
                        idx_wait(1 - bslot)
                        data_issue(1 - bslot)

                    data_wait(bslot)
                    compute(t, bslot)

                    @pl.when(t + 2 < nch)
                    def _():
                        idx_fetch(t + 2, bslot)
            return 0

        lax.fori_loop(0, 20, outer, 0)

    return body(ab_tab, src, dst)


def _p1a_body(cnt0, cnt1, x, demb, wq, bq, wk, bk, wv, bv, wsk, bsk, g, b,
              x_in_o, q_o, k_o, v0_o, v1_o, hskip_o):
    deg = (cnt0[:, 0:1] + cnt1[:, 0:1]).astype(jnp.int32)
    deg = jnp.clip(deg, 0, MAXDEG - 1)
    iota = jax.lax.broadcasted_iota(jnp.int32, (deg.shape[0], MAXDEG), 1)
    onehot = (iota == deg).astype(jnp.float32)
    x_in = x[...] + jnp.dot(onehot, demb[...],
                            preferred_element_type=jnp.float32)
    h = _ln(x_in, g[...], b[...])
    q = jnp.dot(h, wq[...], preferred_element_type=jnp.float32) + bq[...]
    k = jnp.dot(h, wk[...], preferred_element_type=jnp.float32) + bk[...]
    v = jnp.dot(h, wv[...], preferred_element_type=jnp.float32) + bv[...]
    hs = jnp.dot(h, wsk[...], preferred_element_type=jnp.float32) + bsk[...]
    x_in_o[...] = x_in
    q_o[...] = q
    k_o[...] = k
    v0_o[...] = v[:, :128]
    v1_o[...] = v[:, 128:]
    hskip_o[...] = hs


def _p1a(cnt0, cnt1, x, demb, wq, bq, wk, bk, wv, bv, wsk, bsk, g, b):
    nb = pl.BlockSpec((BN, 256), lambda i: (i, 0))
    nh = pl.BlockSpec((BN, 128), lambda i: (i, 0))
    n16 = pl.BlockSpec((BN, 16), lambda i: (i, 0))
    full = lambda s: pl.BlockSpec(s, lambda i: (0,) * len(s))
    return pl.pallas_call(
        _p1a_body,
        grid=(N // BN,),
        in_specs=[pl.BlockSpec((BN, 128), lambda i: (i, 0)),
                  pl.BlockSpec((BN, 128), lambda i: (N // BN + i, 0)),
                  nb, full((MAXDEG, 256)),
                  full((256, 256)), full((1, 256)),
                  full((256, 256)), full((1, 256)),
                  full((256, 256)), full((1, 256)),
                  full((256, 256)), full((1, 256)),
                  full((1, 256)), full((1, 256))],
        out_specs=[nb, nb, nb, nh, nh, nb],
        out_shape=[jax.ShapeDtypeStruct((N, 256), jnp.float32),
                   jax.ShapeDtypeStruct((N, 256), jnp.float32),
                   jax.ShapeDtypeStruct((N, 256), jnp.float32),
                   jax.ShapeDtypeStruct((N, 128), jnp.float32),
                   jax.ShapeDtypeStruct((N, 128), jnp.float32),
                   jax.ShapeDtypeStruct((N, 256), jnp.float32)],
        interpret=_INTERP,
    )(cnt0, cnt1, x, demb, wq, bq, wk, bk, wv, bv, wsk, bsk, g, b)


def _p1b_body(ea, we, ef_o):
    ef_o[...] = jnp.dot(ea[...], we[...], preferred_element_type=jnp.float32)


def _p1b(ea, we):
    nblk = E // BE
    return pl.pallas_call(
        _p1b_body,
        grid=(2, nblk),
        in_specs=[pl.BlockSpec((BE, EDIM), lambda j, i: (i, 0)),
                  pl.BlockSpec((EDIM, 128), lambda j, i: (0, j))],
        out_specs=pl.BlockSpec((BE, 128), lambda j, i: (j * nblk + i, 0)),
        out_shape=jax.ShapeDtypeStruct((2 * E, 128), jnp.float32),
        interpret=_INTERP,
    )(ea, we)


def _p5_body(a0, a1, den0, den1, x_in, hskip, g, b, w1, b1, w2, b2,
             eng, enb, wab, x_new_o, ab_o):
    bn = a0.shape[0]
    attn = jnp.concatenate([a0[...], a1[...]], axis=-1)
    dh = den0[:, 0:8] + den1[:, 0:8]
    dfull = jnp.reshape(jnp.broadcast_to(dh[:, :, None], (bn, 8, 32)),
                        (bn, 256))
    attn = attn / (dfull + 1e-16)
    x_res = x_in[...] + attn + hskip[...]
    hf = _ln(x_res, g[...], b[...])
    up = jnp.dot(hf, w1[...], preferred_element_type=jnp.float32) + b1[...]
    act = _gelu(up)
    ffn = jnp.dot(act, w2[...], preferred_element_type=jnp.float32) + b2[...]
    x_new = x_res + ffn
    xn = _ln(x_new, eng[...], enb[...])
    ab = jnp.dot(xn, wab[...], preferred_element_type=jnp.float32)
    x_new_o[...] = x_new
    ab_o[...] = jnp.concatenate(
        [ab, jnp.zeros((bn, 64), jnp.float32)], axis=-1)


def _p5(a0, a1, den0, den1, x_in, hskip, g, b, w1, b1, w2, b2, eng, enb,
        wab):
    nb = pl.BlockSpec((BN, 256), lambda i: (i, 0))
    nh = pl.BlockSpec((BN, 128), lambda i: (i, 0))
    n16 = pl.BlockSpec((BN, 16), lambda i: (i, 0))
    full = lambda s: pl.BlockSpec(s, lambda i: (0,) * len(s))
    return pl.pallas_call(
        _p5_body,
        grid=(N // BN,),
        in_specs=[pl.BlockSpec((BN, 128), lambda i: (i, 0)),
                  pl.BlockSpec((BN, 128), lambda i: (N // BN + i, 0)),
                  pl.BlockSpec((BN, 128), lambda i: (i, 0)),
                  pl.BlockSpec((BN, 128), lambda i: (N // BN + i, 0)),
                  nb, nb, full((1, 256)), full((1, 256)),
                  full((256, 1024)), full((1, 1024)),
                  full((1024, 256)), full((1, 256)),
                  full((1, 256)), full((1, 256)), full((256, 64))],
        out_specs=[nb, nh],
        out_shape=[jax.ShapeDtypeStruct((N, 256), jnp.float32),
                   jax.ShapeDtypeStruct((N, 128), jnp.float32)],
        interpret=_INTERP,
    )(a0, a1, den0, den1, x_in, hskip, g, b, w1, b1, w2, b2, eng, enb, wab)


def _p7_body(gsum, ea, wc, eb, eg, ebb, e_new_o):
    ct = jnp.dot(ea[...], wc[...], preferred_element_type=jnp.float32)
    raw = jax.nn.relu(gsum[...] + ct + eb[...])
    delta = raw[:, :EDIM]
    gate = raw[:, EDIM:]
    tmp = ea[...] + jax.nn.sigmoid(gate) * delta
    e_new_o[...] = _ln(tmp, eg[...], ebb[...])


def _p7(gsum, ea, wc, eb, eg, ebb):
    full = lambda s: pl.BlockSpec(s, lambda i: (0,) * len(s))
    return pl.pallas_call(
        _p7_body,
        grid=(E // BE,),
        in_specs=[pl.BlockSpec((BE, 32), lambda i: (i, 0)),
                  pl.BlockSpec((BE, EDIM), lambda i: (i, 0)),
                  full((EDIM, 32)), full((1, 32)),
                  full((1, EDIM)), full((1, EDIM))],
        out_specs=pl.BlockSpec((BE, EDIM), lambda i: (i, 0)),
        out_shape=jax.ShapeDtypeStruct((E, EDIM), jnp.float32),
        interpret=_INTERP,
    )(gsum, ea, wc, eb, eg, ebb)


def kernel(x, edge_index, edge_attr, norm_g, norm_b, deg_emb, Wq, bq, Wk, bk,
           Wv, bv, We, Wskip, bskip, W1, b1, W2, b2, eu_ng, eu_nb, eu_W,
           eu_b, eu_eg, eu_eb):
    src = edge_index[0]
    dst = edge_index[1]
    r2 = lambda v: v.reshape(1, -1)
    zeros128 = jnp.zeros((N, 128), jnp.float32)
    ones128 = jnp.ones((BSC, 128), jnp.float32)

    cnt = _p0(src, zeros128, ones128)

    x_in, q, k, v0, v1, hskip = _p1a(
        cnt, cnt, x, deg_emb, Wq, r2(bq), Wk, r2(bk), Wv, r2(bv),
        Wskip, r2(bskip), r2(norm_g), r2(norm_b))
    ecat = _p1b(edge_attr, We)

    ae = _p2a(q, k, ecat, src, dst)
    den = _p2b(ae, dst, zeros128)

    attn = _p4(v0, v1, ecat, ae, src, dst, zeros128)

    wab = jnp.concatenate([eu_W[:256], eu_W[256:512]], axis=1)
    x_new, ab = _p5(attn, attn, den, den, x_in, hskip, r2(norm_g), r2(norm_b),
                    W1, r2(b1), W2, r2(b2), r2(eu_ng), r2(eu_nb), wab)

    gsum = _p6(ab, src, dst)

    e_new = _p7(gsum, edge_attr, eu_W[512:], r2(eu_b), r2(eu_eg), r2(eu_eb))
    return (x_new, e_new)
